# Initial kernel scaffold; baseline (speedup 1.0000x reference)
import jax
import jax.numpy as jnp
from jax.experimental import pallas as pl


def kernel(x, edge_index, edge_attr, batch, params):
    raise NotImplementedError("write your pallas kernel here")


# speedup vs baseline: 2.1806x; 2.1806x over previous
import functools

import jax
import jax.numpy as jnp
from jax import lax
from jax.experimental import pallas as pl
from jax.experimental.pallas import tpu as pltpu
from jax.experimental.pallas import tpu_sc as plsc

N, E, F, ED, G, H = 10000, 320000, 128, 16, 128, 64
NEG = 0.01
H3 = 3 * H

NC, NS = 2, 16
NW = NC * NS
C = 128
TCH = E // C
CHB = TCH // NW
CHR = TCH - CHB * NW


def _lrelu(v):
    return jnp.where(v >= 0, v, v * NEG)


def _elu(v):
    return jnp.where(v > 0, v, jnp.exp(v) - 1.0)


def _gru(h, hh, wih_t, whh_t, bih, bhh):
    gi = jnp.dot(h, wih_t, preferred_element_type=jnp.float32) + bih
    gh = jnp.dot(hh, whh_t, preferred_element_type=jnp.float32) + bhh
    r = jax.nn.sigmoid(gi[:, :H] + gh[:, :H])
    z = jax.nn.sigmoid(gi[:, H:2 * H] + gh[:, H:2 * H])
    n = jnp.tanh(gi[:, 2 * H:] + r * gh[:, 2 * H:])
    return (1.0 - z) * n + z * hh


_sc_mesh = plsc.VectorSubcoreMesh(core_axis_name="c", subcore_axis_name="s",
                                  num_cores=NC, num_subcores=NS)


@functools.partial(
    pl.kernel,
    out_type=jax.ShapeDtypeStruct((E, F), jnp.float32),
    mesh=_sc_mesh,
    compiler_params=pltpu.CompilerParams(needs_layout_passes=False),
    scratch_types=[
        pltpu.VMEM((C,), jnp.int32),
        pltpu.VMEM((C, F), jnp.float32),
        pltpu.SemaphoreType.DMA,
    ],
)
def _gather_sc(tab_hbm, idx_hbm, out_hbm, idx_v, rows_v, sem):
    cid = lax.axis_index("c")
    sid = lax.axis_index("s")
    wid = sid * NC + cid
    nch = jnp.where(wid < CHR, CHB + 1, CHB)

    def _chunk(j, carry):
        base = (wid + j * NW) * C
        pltpu.sync_copy(idx_hbm.at[pl.ds(base, C)], idx_v)
        pltpu.async_copy(tab_hbm.at[idx_v], rows_v, sem).wait()
        pltpu.sync_copy(rows_v, out_hbm.at[pl.ds(base, C)])
        return carry

    lax.fori_loop(0, nch, _chunk, 0)


def _prep_body(x_ref, w1_ref, b1_ref, wg1a_ref, wg2_ref, attr_ref,
               x1_ref, nt_ref, srd_ref):
    x1 = _lrelu(jnp.dot(x_ref[...], w1_ref[...],
                        preferred_element_type=jnp.float32) + b1_ref[...])
    x1_ref[...] = x1
    xw1 = jnp.dot(x1, wg1a_ref[...], preferred_element_type=jnp.float32)
    x2 = jnp.dot(x1, wg2_ref[...], preferred_element_type=jnp.float32)
    nt_ref[...] = jnp.concatenate([xw1, x2], axis=1)
    srd_ref[...] = jnp.sum(x1 * attr_ref[...], axis=1)[:, None]


def _prep(x, w1, b1, wg1a, wg2, attr):
    B = 2000
    return pl.pallas_call(
        _prep_body,
        grid=(N // B,),
        in_specs=[
            pl.BlockSpec((B, F), lambda i: (i, 0)),
            pl.BlockSpec((F, H), lambda i: (0, 0)),
            pl.BlockSpec((1, H), lambda i: (0, 0)),
            pl.BlockSpec((H, H), lambda i: (0, 0)),
            pl.BlockSpec((H, H), lambda i: (0, 0)),
            pl.BlockSpec((1, H), lambda i: (0, 0)),
        ],
        out_specs=[
            pl.BlockSpec((B, H), lambda i: (i, 0)),
            pl.BlockSpec((B, 2 * H), lambda i: (i, 0)),
            pl.BlockSpec((B, 1), lambda i: (i, 0)),
        ],
        out_shape=[
            jax.ShapeDtypeStruct((N, H), jnp.float32),
            jax.ShapeDtypeStruct((N, 2 * H), jnp.float32),
            jax.ShapeDtypeStruct((N, 1), jnp.float32),
        ],
    )(x, w1, b1, wg1a, wg2, attr)


def _edge1_body(rows_ref, ea_ref, srdj_ref, wg1b_ref, attl_ref,
                msg_ref, w_ref):
    rows = rows_ref[...]
    eaw = jnp.dot(ea_ref[...], wg1b_ref[...],
                  preferred_element_type=jnp.float32)
    v = _lrelu(rows[:, :H] + eaw)
    l = jnp.sum(v * attl_ref[...], axis=1) + srdj_ref[...][:, 0]
    w = jnp.exp(_lrelu(l))
    msg_ref[...] = rows[:, H:] * w[:, None]
    w_ref[...] = w[:, None]


def _edge1(rows, edge_attr, srdj, wg1b, attl):
    B = 8000
    return pl.pallas_call(
        _edge1_body,
        grid=(E // B,),
        in_specs=[
            pl.BlockSpec((B, F), lambda i: (i, 0)),
            pl.BlockSpec((B, ED), lambda i: (i, 0)),
            pl.BlockSpec((B, 1), lambda i: (i, 0)),
            pl.BlockSpec((ED, H), lambda i: (0, 0)),
            pl.BlockSpec((1, H), lambda i: (0, 0)),
        ],
        out_specs=[
            pl.BlockSpec((B, H), lambda i: (i, 0)),
            pl.BlockSpec((B, 1), lambda i: (i, 0)),
        ],
        out_shape=[
            jax.ShapeDtypeStruct((E, H), jnp.float32),
            jax.ShapeDtypeStruct((E, 1), jnp.float32),
        ],
    )(rows, edge_attr, srdj, wg1b, attl)


def _edge2_body(rows_ref, ssj_ref, sdj_ref, msg_ref, w_ref):
    rows = rows_ref[...]
    w = jnp.exp(_lrelu(ssj_ref[...][:, 0] + sdj_ref[...][:, 0]))
    msg_ref[...] = rows[:, :H] * w[:, None]
    w_ref[...] = w[:, None]


def _edge2(rows, ssj, sdj):
    B = 8000
    return pl.pallas_call(
        _edge2_body,
        grid=(E // B,),
        in_specs=[
            pl.BlockSpec((B, F), lambda i: (i, 0)),
            pl.BlockSpec((B, 1), lambda i: (i, 0)),
            pl.BlockSpec((B, 1), lambda i: (i, 0)),
        ],
        out_specs=[
            pl.BlockSpec((B, H), lambda i: (i, 0)),
            pl.BlockSpec((B, 1), lambda i: (i, 0)),
        ],
        out_shape=[
            jax.ShapeDtypeStruct((E, H), jnp.float32),
            jax.ShapeDtypeStruct((E, 1), jnp.float32),
        ],
    )(rows, ssj, sdj)


def _gru1_body(acc_ref, s_ref, xp_ref, gb_ref, wih_ref, whh_ref, bih_ref,
               bhh_ref, gatw_ref, asrc_ref, adst_ref,
               xn_ref, xl_ref, ss_ref, sd_ref):
    h = acc_ref[...] / (s_ref[...] + 1e-16) + gb_ref[...]
    h = _elu(h)
    xp = xp_ref[...]
    xn = jnp.maximum(
        _gru(h, xp, wih_ref[...], whh_ref[...], bih_ref[...], bhh_ref[...]),
        0.0)
    xn_ref[...] = xn
    xl = jnp.dot(xn, gatw_ref[...], preferred_element_type=jnp.float32)
    xl_ref[...] = jnp.concatenate([xl, jnp.zeros_like(xl)], axis=1)
    ss_ref[...] = jnp.sum(xl * asrc_ref[...], axis=1)[:, None]
    sd_ref[...] = jnp.sum(xl * adst_ref[...], axis=1)[:, None]


def _gru1(acc, s, xp, gb, wih_t, whh_t, bih, bhh, gatw, asrc, adst):
    B = 2000
    return pl.pallas_call(
        _gru1_body,
        grid=(N // B,),
        in_specs=[
            pl.BlockSpec((B, H), lambda i: (i, 0)),
            pl.BlockSpec((B, 1), lambda i: (i, 0)),
            pl.BlockSpec((B, H), lambda i: (i, 0)),
            pl.BlockSpec((1, H), lambda i: (0, 0)),
            pl.BlockSpec((H, H3), lambda i: (0, 0)),
            pl.BlockSpec((H, H3), lambda i: (0, 0)),
            pl.BlockSpec((1, H3), lambda i: (0, 0)),
            pl.BlockSpec((1, H3), lambda i: (0, 0)),
            pl.BlockSpec((H, H), lambda i: (0, 0)),
            pl.BlockSpec((1, H), lambda i: (0, 0)),
            pl.BlockSpec((1, H), lambda i: (0, 0)),
        ],
        out_specs=[
            pl.BlockSpec((B, H), lambda i: (i, 0)),
            pl.BlockSpec((B, 2 * H), lambda i: (i, 0)),
            pl.BlockSpec((B, 1), lambda i: (i, 0)),
            pl.BlockSpec((B, 1), lambda i: (i, 0)),
        ],
        out_shape=[
            jax.ShapeDtypeStruct((N, H), jnp.float32),
            jax.ShapeDtypeStruct((N, 2 * H), jnp.float32),
            jax.ShapeDtypeStruct((N, 1), jnp.float32),
            jax.ShapeDtypeStruct((N, 1), jnp.float32),
        ],
    )(acc, s, xp, gb, wih_t, whh_t, bih, bhh, gatw, asrc, adst)


def _gru2_body(acc_ref, s_ref, xp_ref, gb_ref, wih_ref, whh_ref, bih_ref,
               bhh_ref, molw_ref, masrc_ref, xf_ref, xm_ref, asrc_ref):
    h = acc_ref[...] / (s_ref[...] + 1e-16) + gb_ref[...]
    h = _elu(h)
    xp = xp_ref[...]
    xf = jnp.maximum(
        _gru(h, xp, wih_ref[...], whh_ref[...], bih_ref[...], bhh_ref[...]),
        0.0)
    xf_ref[...] = xf
    xm = jnp.dot(xf, molw_ref[...], preferred_element_type=jnp.float32)
    xm_ref[...] = xm
    asrc_ref[...] = jnp.sum(xm * masrc_ref[...], axis=1)[:, None]


def _gru2(acc, s, xp, gb, wih_t, whh_t, bih, bhh, molw, masrc):
    B = 2000
    return pl.pallas_call(
        _gru2_body,
        grid=(N // B,),
        in_specs=[
            pl.BlockSpec((B, H), lambda i: (i, 0)),
            pl.BlockSpec((B, 1), lambda i: (i, 0)),
            pl.BlockSpec((B, H), lambda i: (i, 0)),
            pl.BlockSpec((1, H), lambda i: (0, 0)),
            pl.BlockSpec((H, H3), lambda i: (0, 0)),
            pl.BlockSpec((H, H3), lambda i: (0, 0)),
            pl.BlockSpec((1, H3), lambda i: (0, 0)),
            pl.BlockSpec((1, H3), lambda i: (0, 0)),
            pl.BlockSpec((H, H), lambda i: (0, 0)),
            pl.BlockSpec((1, H), lambda i: (0, 0)),
        ],
        out_specs=[
            pl.BlockSpec((B, H), lambda i: (i, 0)),
            pl.BlockSpec((B, H), lambda i: (i, 0)),
            pl.BlockSpec((B, 1), lambda i: (i, 0)),
        ],
        out_shape=[
            jax.ShapeDtypeStruct((N, H), jnp.float32),
            jax.ShapeDtypeStruct((N, H), jnp.float32),
            jax.ShapeDtypeStruct((N, 1), jnp.float32),
        ],
    )(acc, s, xp, gb, wih_t, whh_t, bih, bhh, molw, masrc)


def _mol_body(xf_ref, xm_ref, asrc_ref, batch_ref, molw_ref, madst_ref,
              mb_ref, wih_ref, whh_ref, bih_ref, bhh_ref, w2_ref, b2_ref,
              out_ref):
    bt = batch_ref[0]
    gi = lax.broadcasted_iota(jnp.int32, (G, N), 0)
    mtb = gi == bt[None, :]
    mt = mtb.astype(jnp.float32)
    xf = xf_ref[...]
    xm = xm_ref[...]
    a0 = asrc_ref[0]
    out = jnp.maximum(jnp.dot(mt, xf, preferred_element_type=jnp.float32), 0.0)
    for _ in range(2):
        om = jnp.dot(out, molw_ref[...], preferred_element_type=jnp.float32)
        adg = jnp.sum(om * madst_ref[...], axis=1)
        ad_n = jnp.sum(jnp.where(mtb, adg[:, None], 0.0), axis=0)
        a = _lrelu(a0 + ad_n)
        m_g = jnp.max(jnp.where(mtb, a[None, :], -jnp.inf), axis=1)
        m_g = jnp.where(jnp.isfinite(m_g), m_g, 0.0)
        m_n = jnp.sum(jnp.where(mtb, m_g[:, None], 0.0), axis=0)
        e_n = jnp.exp(a - m_n)
        s_g = jnp.sum(jnp.where(mtb, e_n[None, :], 0.0), axis=1)
        s_n = jnp.sum(jnp.where(mtb, s_g[:, None], 0.0), axis=0)
        alpha = e_n / (s_n + 1e-16)
        h = jnp.dot(mt, xm * alpha[:, None],
                    preferred_element_type=jnp.float32) + mb_ref[...]
        h = _elu(h)
        out = jnp.maximum(
            _gru(h, out, wih_ref[...], whh_ref[...], bih_ref[...],
                 bhh_ref[...]), 0.0)
    out_ref[...] = (jnp.sum(out * w2_ref[...], axis=1, keepdims=True)
                    + b2_ref[...])


def _mol(xf, xm, asrc, batch2d, molw, madst, mb, wih_t, whh_t, bih, bhh,
         w2, b2):
    return pl.pallas_call(
        _mol_body,
        out_shape=jax.ShapeDtypeStruct((G, 1), jnp.float32),
    )(xf, xm, asrc, batch2d, molw, madst, mb, wih_t, whh_t, bih, bhh, w2, b2)


def kernel(x, edge_index, edge_attr, batch, params):
    p = params
    src = edge_index[0]
    dst = edge_index[1]
    wg1a = p['g_lin1_W'][:H]
    wg1b = p['g_lin1_W'][H:]
    row = lambda v: v.reshape(1, -1)

    x1, nodetab, srd2d = _prep(x, p['lin1_W'], row(p['lin1_b']), wg1a,
                               p['g_lin2_W'], row(p['g_att_r']))
    rows1 = _gather_sc(nodetab, src)
    srdj = srd2d[dst]
    msg1, w1 = _edge1(rows1, edge_attr, srdj, wg1b, row(p['g_att_l']))
    acc1 = jax.ops.segment_sum(msg1, dst, num_segments=N)
    s1 = jax.ops.segment_sum(w1, dst, num_segments=N)
    xn, xl, ss2d, sd2d = _gru1(acc1, s1, x1, row(p['g_bias']),
                               p['gru1_Wih'].T, p['gru1_Whh'].T,
                               row(p['gru1_bih']), row(p['gru1_bhh']),
                               p['gat_W'], row(p['gat_att_src']),
                               row(p['gat_att_dst']))
    rows2 = _gather_sc(xl, src)
    msg2, w2 = _edge2(rows2, ss2d[src], sd2d[dst])
    acc2 = jax.ops.segment_sum(msg2, dst, num_segments=N)
    s2 = jax.ops.segment_sum(w2, dst, num_segments=N)
    xf, xm, asrc = _gru2(acc2, s2, xn, row(p['gat_bias']), p['gru2_Wih'].T,
                         p['gru2_Whh'].T, row(p['gru2_bih']),
                         row(p['gru2_bhh']), p['mol_W'],
                         row(p['mol_att_src']))
    out = _mol(xf, xm, asrc.reshape(1, N), batch.reshape(1, N), p['mol_W'],
               row(p['mol_att_dst']), row(p['mol_bias']), p['mol_Wih'].T,
               p['mol_Whh'].T, row(p['mol_bih']), row(p['mol_bhh']),
               p['lin2_W'].T, p['lin2_b'].reshape(1, 1))
    return out
